# baseline (speedup 1.0000x reference)
import functools

import jax
import jax.numpy as jnp
from jax import lax
from jax.experimental import pallas as pl
from jax.experimental.pallas import tpu as pltpu
from jax.experimental.pallas import tpu_sc as plsc

NN = 10000
DD = 128
HH = 128
CC = 40

NC = 2
NS = 16
NWK = NC * NS
CHUNK = 128
NPAD = 10240
ROWS_PER_TILE = NPAD // NS

NROW = 2
NIDX = 8
GA = 1
SLAG = NROW - GA
DNROW = 4
DNIDX = 8
CPW_A_FRAC, CPW_DEN = 9, 10


def _mesh():
    return plsc.VectorSubcoreMesh(core_axis_name="c", subcore_axis_name="s")


def _zero_fill(buf, n_rows):
    def zero(g, carry):
        for j in range(DD // 16):
            buf[g, pl.ds(j * 16, 16)] = jnp.zeros((16,), jnp.float32)
        return carry

    lax.fori_loop(0, n_rows, zero, 0)


def _sc_degree(dst_r, cpw):
    @functools.partial(
        pl.kernel,
        out_type=jax.ShapeDtypeStruct((NC, NPAD, DD), jnp.float32),
        mesh=_mesh(),
        scratch_types=(
            [pltpu.VMEM((CHUNK, DD), jnp.float32)]
            + [pltpu.VMEM((CHUNK,), jnp.int32)] * DNIDX
            + [pltpu.SemaphoreType.DMA] * DNIDX
            + [pltpu.SemaphoreType.DMA] * DNROW
            + [pltpu.VMEM_SHARED((NPAD, DD), jnp.float32)]
        ),
    )
    def k(dst_hbm, out_hbm, *rest):
        ones_v = rest[0]
        idxd = rest[1:1 + DNIDX]
        si = rest[1 + DNIDX:1 + 2 * DNIDX]
        ss = rest[1 + 2 * DNIDX:1 + 2 * DNIDX + DNROW]
        acc = rest[1 + 2 * DNIDX + DNROW]

        c = lax.axis_index("c")
        s = lax.axis_index("s")
        wid = s * NC + c

        _zero_fill(ones_v, CHUNK)
        for q in range(DNIDX):
            idxd[q][pl.ds(0, 16)] = jnp.zeros((16,), jnp.int32)
        for t in range(ROWS_PER_TILE // CHUNK):
            pltpu.sync_copy(ones_v, acc.at[pl.ds(s * ROWS_PER_TILE + t * CHUNK, CHUNK)])

        def fill(g, carry):
            for j in range(DD // 16):
                ones_v[g, pl.ds(j * 16, 16)] = jnp.full((16,), 1.0, jnp.float32)
            return carry

        lax.fori_loop(0, CHUNK, fill, 0)
        plsc.subcore_barrier()

        def idx_start(t, q):
            pltpu.async_copy(dst_hbm.at[wid].at[pl.ds(t * CHUNK, CHUNK)], idxd[q], si[q])

        def idx_wait(q):
            pltpu.make_async_copy(dst_hbm.at[wid].at[pl.ds(0, CHUNK)], idxd[q], si[q]).wait()

        def sc_start(q, b):
            pltpu.async_copy(ones_v, acc.at[idxd[q]], ss[b], add=True)

        def sc_wait(b):
            pltpu.make_async_copy(ones_v, acc.at[idxd[0]], ss[b]).wait()

        for q in range(DNROW):
            idx_start(q, q)

        def body(i, carry):
            for kk in range(DNIDX):
                t = i * DNIDX + kk
                b = kk % DNROW

                @pl.when(t >= DNROW)
                def _():
                    sc_wait(b)

                @pl.when(t + DNROW < cpw)
                def _():
                    idx_start(t + DNROW, (kk + DNROW) % DNIDX)

                idx_wait(kk)
                sc_start(kk, b)
            return carry

        lax.fori_loop(0, cpw // DNIDX, body, 0)
        for b in range(DNROW):
            sc_wait(b)
        plsc.subcore_barrier()

        for t in range(ROWS_PER_TILE // CHUNK):
            r0 = s * ROWS_PER_TILE + t * CHUNK
            pltpu.sync_copy(acc.at[pl.ds(r0, CHUNK)], ones_v)
            pltpu.sync_copy(ones_v, out_hbm.at[c].at[pl.ds(r0, CHUNK)])

    return k(dst_r)


def _sc_segment_sum(hp, src_f, dst_f, cpw_a, cpw_b):
    @functools.partial(
        pl.kernel,
        out_type=jax.ShapeDtypeStruct((NC, NPAD, DD), jnp.float32),
        mesh=_mesh(),
        scratch_types=(
            [pltpu.VMEM((CHUNK,), jnp.int32)] * NIDX
            + [pltpu.VMEM((CHUNK,), jnp.int32)] * NIDX
            + [pltpu.VMEM((CHUNK, DD), jnp.float32)] * NROW
            + [pltpu.SemaphoreType.DMA] * NIDX
            + [pltpu.SemaphoreType.DMA] * NROW
            + [pltpu.SemaphoreType.DMA] * NROW
            + [pltpu.VMEM_SHARED((NPAD, DD), jnp.float32)]
        ),
    )
    def k(hp_hbm, src_hbm, dst_hbm, out_hbm, *rest):
        idxs = rest[0:NIDX]
        idxd = rest[NIDX:2 * NIDX]
        rows = rest[2 * NIDX:2 * NIDX + NROW]
        si = rest[2 * NIDX + NROW:3 * NIDX + NROW]
        sg = rest[3 * NIDX + NROW:3 * NIDX + 2 * NROW]
        ss = rest[3 * NIDX + 2 * NROW:3 * NIDX + 3 * NROW]
        acc = rest[3 * NIDX + 3 * NROW]

        c = lax.axis_index("c")
        s = lax.axis_index("s")
        nch = jnp.where(c == 0, cpw_a, cpw_b)
        base_ch = jnp.where(c == 0, s * cpw_a, NS * cpw_a + s * cpw_b)

        _zero_fill(rows[0], CHUNK)
        for b in range(1, NROW):
            rows[b][0, pl.ds(0, 16)] = jnp.zeros((16,), jnp.float32)
        for q in range(NIDX):
            idxs[q][pl.ds(0, 16)] = jnp.zeros((16,), jnp.int32)
            idxd[q][pl.ds(0, 16)] = jnp.zeros((16,), jnp.int32)
        for t in range(ROWS_PER_TILE // CHUNK):
            pltpu.sync_copy(rows[0], acc.at[pl.ds(s * ROWS_PER_TILE + t * CHUNK, CHUNK)])
        plsc.subcore_barrier()

        def idx_start(t, q):
            e0 = (base_ch + t) * CHUNK
            pltpu.async_copy(src_hbm.at[pl.ds(e0, CHUNK)], idxs[q], si[q])
            pltpu.async_copy(dst_hbm.at[pl.ds(e0, CHUNK)], idxd[q], si[q])

        def idx_wait(q):
            pltpu.make_async_copy(src_hbm.at[pl.ds(0, CHUNK)], idxs[q], si[q]).wait()
            pltpu.make_async_copy(dst_hbm.at[pl.ds(0, CHUNK)], idxd[q], si[q]).wait()

        def g_start(q, b):
            pltpu.async_copy(hp_hbm.at[idxs[q]], rows[b], sg[b])

        def g_wait(q, b):
            pltpu.make_async_copy(hp_hbm.at[idxs[q]], rows[b], sg[b]).wait()

        def sc_start(q, b):
            pltpu.async_copy(rows[b], acc.at[idxd[q]], ss[b], add=True)

        def sc_wait(b):
            pltpu.make_async_copy(rows[b], acc.at[idxd[0]], ss[b]).wait()

        @pl.when(nch > 0)
        def _():
            for q in range(NROW):
                idx_start(q, q)
            for t in range(GA):
                idx_wait(t)
                g_start(t, t)

        def body(i, carry):
            for kk in range(NIDX):
                t = i * NIDX + kk
                b = kk % NROW

                g_wait(kk, b)
                sc_start(kk, b)

                @pl.when(t + NROW < nch)
                def _():
                    idx_start(t + NROW, (kk + NROW) % NIDX)

                @pl.when((t + GA < nch) & (t >= SLAG))
                def _():
                    sc_wait((kk + GA) % NROW)

                @pl.when(t + GA < nch)
                def _():
                    idx_wait((kk + GA) % NIDX)
                    g_start((kk + GA) % NIDX, (kk + GA) % NROW)
            return carry

        lax.fori_loop(0, nch // NIDX, body, 0)

        @pl.when(nch > 0)
        def _():
            for b in range(NROW):
                sc_wait(b)

        plsc.subcore_barrier()

        for t in range(ROWS_PER_TILE // CHUNK):
            r0 = s * ROWS_PER_TILE + t * CHUNK
            pltpu.sync_copy(acc.at[pl.ds(r0, CHUNK)], rows[0])
            pltpu.sync_copy(rows[0], out_hbm.at[c].at[pl.ds(r0, CHUNK)])

    return k(hp, src_f, dst_f)


_BLK = 1000


def _dinv_block(deg2_ref):
    deg = deg2_ref[0, :, :1] + deg2_ref[1, :, :1] + 1.0
    return lax.rsqrt(jnp.maximum(deg, 1.0))


def _tc_first_body(deg2_ref, x_ref, w1_ref, out_ref):
    dinv = _dinv_block(deg2_ref)
    h = jnp.dot(x_ref[...], w1_ref[...], preferred_element_type=jnp.float32)
    out_ref[...] = h * dinv


def _tc_mid_body(deg2_ref, agg_ref, hp_ref, b_ref, w_ref, out_ref):
    dinv = _dinv_block(deg2_ref)
    z = dinv * (agg_ref[0] + agg_ref[1] + hp_ref[...]) + b_ref[...]
    z = jnp.where(z > 0.0, z, jnp.exp(z) - 1.0)
    out_ref[...] = jnp.dot(z, w_ref[...], preferred_element_type=jnp.float32) * dinv


def _tc_final_body(deg2_ref, agg_ref, hp_ref, b_ref, w_ref, b3_ref, out_ref):
    dinv = _dinv_block(deg2_ref)
    z = dinv * (agg_ref[0] + agg_ref[1] + hp_ref[...]) + b_ref[...]
    z = jnp.where(z > 0.0, z, jnp.exp(z) - 1.0)
    logits = jnp.dot(z, w_ref[...], preferred_element_type=jnp.float32) + b3_ref[...]
    m = jnp.max(logits, axis=1, keepdims=True)
    e = jnp.exp(logits - m)
    lse = jnp.log(jnp.sum(e, axis=1, keepdims=True)) + m
    out_ref[...] = logits - lse


def _deg2_spec():
    return pl.BlockSpec((NC, _BLK, DD), lambda i: (0, i, 0))


def _agg_spec():
    return pl.BlockSpec((NC, _BLK, DD), lambda i: (0, i, 0))


def _row_spec(d):
    return pl.BlockSpec((_BLK, d), lambda i: (i, 0))


def _full_spec(r, c):
    return pl.BlockSpec((r, c), lambda i: (0, 0))


def _tc_first(deg2, x, W1):
    return pl.pallas_call(
        _tc_first_body,
        grid=(NN // _BLK,),
        in_specs=[_deg2_spec(), _row_spec(DD), _full_spec(DD, HH)],
        out_specs=_row_spec(HH),
        out_shape=jax.ShapeDtypeStruct((NN, HH), jnp.float32),
    )(deg2, x, W1)


def _tc_mid(deg2, agg, hp, b1, W2):
    return pl.pallas_call(
        _tc_mid_body,
        grid=(NN // _BLK,),
        in_specs=[_deg2_spec(), _agg_spec(), _row_spec(HH),
                  _full_spec(1, HH), _full_spec(HH, HH)],
        out_specs=_row_spec(HH),
        out_shape=jax.ShapeDtypeStruct((NN, HH), jnp.float32),
    )(deg2, agg, hp, b1, W2)


def _tc_final(deg2, agg, hp, b2, W3, b3):
    return pl.pallas_call(
        _tc_final_body,
        grid=(NN // _BLK,),
        in_specs=[_deg2_spec(), _agg_spec(), _row_spec(HH),
                  _full_spec(1, HH), _full_spec(HH, CC), _full_spec(1, CC)],
        out_specs=_row_spec(CC),
        out_shape=jax.ShapeDtypeStruct((NN, CC), jnp.float32),
    )(deg2, agg, hp, b2, W3, b3)


def kernel(x, edge_index, W1, b1, W2, b2, W3, b3):
    E = edge_index.shape[1]
    src = edge_index[0].astype(jnp.int32)
    dst = edge_index[1].astype(jnp.int32)

    per = NWK * CHUNK
    cpw = (E + per - 1) // per
    cpw = ((cpw + NIDX - 1) // NIDX) * NIDX
    EP = cpw * per
    pad = EP - E
    if pad:
        src = jnp.concatenate([src, jnp.zeros((pad,), jnp.int32)])
        dst = jnp.concatenate([dst, jnp.full((pad,), NN, jnp.int32)])
    dst_r = dst.reshape(NWK, cpw * CHUNK)

    cpw_a = (CPW_A_FRAC * 2 * cpw // NIDX // CPW_DEN) * NIDX
    cpw_b = 2 * cpw - cpw_a

    deg2 = _sc_degree(dst_r, cpw)
    hp1 = _tc_first(deg2, x, W1)
    agg1 = _sc_segment_sum(hp1, src, dst, cpw_a, cpw_b)
    hp2 = _tc_mid(deg2, agg1, hp1, b1.reshape(1, HH), W2)
    agg2 = _sc_segment_sum(hp2, src, dst, cpw_a, cpw_b)
    return _tc_final(deg2, agg2, hp2, b2.reshape(1, HH), W3, b3.reshape(1, CC))


# speedup vs baseline: 1.0061x; 1.0061x over previous
import functools

import jax
import jax.numpy as jnp
from jax import lax
from jax.experimental import pallas as pl
from jax.experimental.pallas import tpu as pltpu
from jax.experimental.pallas import tpu_sc as plsc

NN = 10000
DD = 128
HH = 128
CC = 40

NC = 2
NS = 16
NWK = NC * NS
CHUNK = 128
NPAD = 10240
ROWS_PER_TILE = NPAD // NS

NROW = 2
NIDX = 8
GA = 1
SLAG = NROW - GA
DNROW = 4
DNIDX = 8
CPW_A_FRAC, CPW_DEN = 19, 20


def _mesh():
    return plsc.VectorSubcoreMesh(core_axis_name="c", subcore_axis_name="s")


def _zero_fill(buf, n_rows):
    def zero(g, carry):
        for j in range(DD // 16):
            buf[g, pl.ds(j * 16, 16)] = jnp.zeros((16,), jnp.float32)
        return carry

    lax.fori_loop(0, n_rows, zero, 0)


def _sc_degree(dst_r, cpw):
    @functools.partial(
        pl.kernel,
        out_type=jax.ShapeDtypeStruct((NC, NPAD, DD), jnp.float32),
        mesh=_mesh(),
        scratch_types=(
            [pltpu.VMEM((CHUNK, DD), jnp.float32)]
            + [pltpu.VMEM((CHUNK,), jnp.int32)] * DNIDX
            + [pltpu.SemaphoreType.DMA] * DNIDX
            + [pltpu.SemaphoreType.DMA] * DNROW
            + [pltpu.VMEM_SHARED((NPAD, DD), jnp.float32)]
        ),
    )
    def k(dst_hbm, out_hbm, *rest):
        ones_v = rest[0]
        idxd = rest[1:1 + DNIDX]
        si = rest[1 + DNIDX:1 + 2 * DNIDX]
        ss = rest[1 + 2 * DNIDX:1 + 2 * DNIDX + DNROW]
        acc = rest[1 + 2 * DNIDX + DNROW]

        c = lax.axis_index("c")
        s = lax.axis_index("s")
        wid = s * NC + c

        _zero_fill(ones_v, CHUNK)
        for q in range(DNIDX):
            idxd[q][pl.ds(0, 16)] = jnp.zeros((16,), jnp.int32)
        for t in range(ROWS_PER_TILE // CHUNK):
            pltpu.sync_copy(ones_v, acc.at[pl.ds(s * ROWS_PER_TILE + t * CHUNK, CHUNK)])

        def fill(g, carry):
            for j in range(DD // 16):
                ones_v[g, pl.ds(j * 16, 16)] = jnp.full((16,), 1.0, jnp.float32)
            return carry

        lax.fori_loop(0, CHUNK, fill, 0)
        plsc.subcore_barrier()

        def idx_start(t, q):
            pltpu.async_copy(dst_hbm.at[wid].at[pl.ds(t * CHUNK, CHUNK)], idxd[q], si[q])

        def idx_wait(q):
            pltpu.make_async_copy(dst_hbm.at[wid].at[pl.ds(0, CHUNK)], idxd[q], si[q]).wait()

        def sc_start(q, b):
            pltpu.async_copy(ones_v, acc.at[idxd[q]], ss[b], add=True)

        def sc_wait(b):
            pltpu.make_async_copy(ones_v, acc.at[idxd[0]], ss[b]).wait()

        for q in range(DNROW):
            idx_start(q, q)

        def body(i, carry):
            for kk in range(DNIDX):
                t = i * DNIDX + kk
                b = kk % DNROW

                @pl.when(t >= DNROW)
                def _():
                    sc_wait(b)

                @pl.when(t + DNROW < cpw)
                def _():
                    idx_start(t + DNROW, (kk + DNROW) % DNIDX)

                idx_wait(kk)
                sc_start(kk, b)
            return carry

        lax.fori_loop(0, cpw // DNIDX, body, 0)
        for b in range(DNROW):
            sc_wait(b)
        plsc.subcore_barrier()

        for t in range(ROWS_PER_TILE // CHUNK):
            r0 = s * ROWS_PER_TILE + t * CHUNK
            pltpu.async_copy(acc.at[pl.ds(r0, CHUNK)], out_hbm.at[c].at[pl.ds(r0, CHUNK)], si[0])
        for t in range(ROWS_PER_TILE // CHUNK):
            r0 = s * ROWS_PER_TILE + t * CHUNK
            pltpu.make_async_copy(acc.at[pl.ds(r0, CHUNK)], out_hbm.at[c].at[pl.ds(r0, CHUNK)], si[0]).wait()

    return k(dst_r)


def _sc_segment_sum(hp, src_f, dst_f, cpw_a, cpw_b):
    @functools.partial(
        pl.kernel,
        out_type=jax.ShapeDtypeStruct((NC, NPAD, DD), jnp.float32),
        mesh=_mesh(),
        scratch_types=(
            [pltpu.VMEM((CHUNK,), jnp.int32)] * NIDX
            + [pltpu.VMEM((CHUNK,), jnp.int32)] * NIDX
            + [pltpu.VMEM((CHUNK, DD), jnp.float32)] * NROW
            + [pltpu.SemaphoreType.DMA] * NIDX
            + [pltpu.SemaphoreType.DMA] * NROW
            + [pltpu.SemaphoreType.DMA] * NROW
            + [pltpu.VMEM_SHARED((NPAD, DD), jnp.float32)]
        ),
    )
    def k(hp_hbm, src_hbm, dst_hbm, out_hbm, *rest):
        idxs = rest[0:NIDX]
        idxd = rest[NIDX:2 * NIDX]
        rows = rest[2 * NIDX:2 * NIDX + NROW]
        si = rest[2 * NIDX + NROW:3 * NIDX + NROW]
        sg = rest[3 * NIDX + NROW:3 * NIDX + 2 * NROW]
        ss = rest[3 * NIDX + 2 * NROW:3 * NIDX + 3 * NROW]
        acc = rest[3 * NIDX + 3 * NROW]

        c = lax.axis_index("c")
        s = lax.axis_index("s")
        nch = jnp.where(c == 0, cpw_a, cpw_b)
        base_ch = jnp.where(c == 0, s * cpw_a, NS * cpw_a + s * cpw_b)

        _zero_fill(rows[0], CHUNK)
        for b in range(1, NROW):
            rows[b][0, pl.ds(0, 16)] = jnp.zeros((16,), jnp.float32)
        for q in range(NIDX):
            idxs[q][pl.ds(0, 16)] = jnp.zeros((16,), jnp.int32)
            idxd[q][pl.ds(0, 16)] = jnp.zeros((16,), jnp.int32)
        for t in range(ROWS_PER_TILE // CHUNK):
            pltpu.sync_copy(rows[0], acc.at[pl.ds(s * ROWS_PER_TILE + t * CHUNK, CHUNK)])
        plsc.subcore_barrier()

        def idx_start(t, q):
            e0 = (base_ch + t) * CHUNK
            pltpu.async_copy(src_hbm.at[pl.ds(e0, CHUNK)], idxs[q], si[q])
            pltpu.async_copy(dst_hbm.at[pl.ds(e0, CHUNK)], idxd[q], si[q])

        def idx_wait(q):
            pltpu.make_async_copy(src_hbm.at[pl.ds(0, CHUNK)], idxs[q], si[q]).wait()
            pltpu.make_async_copy(dst_hbm.at[pl.ds(0, CHUNK)], idxd[q], si[q]).wait()

        def g_start(q, b):
            pltpu.async_copy(hp_hbm.at[idxs[q]], rows[b], sg[b])

        def g_wait(q, b):
            pltpu.make_async_copy(hp_hbm.at[idxs[q]], rows[b], sg[b]).wait()

        def sc_start(q, b):
            pltpu.async_copy(rows[b], acc.at[idxd[q]], ss[b], add=True)

        def sc_wait(b):
            pltpu.make_async_copy(rows[b], acc.at[idxd[0]], ss[b]).wait()

        @pl.when(nch > 0)
        def _():
            for q in range(NROW):
                idx_start(q, q)
            for t in range(GA):
                idx_wait(t)
                g_start(t, t)

        def body(i, carry):
            for kk in range(NIDX):
                t = i * NIDX + kk
                b = kk % NROW

                g_wait(kk, b)
                sc_start(kk, b)

                @pl.when(t + NROW < nch)
                def _():
                    idx_start(t + NROW, (kk + NROW) % NIDX)

                @pl.when((t + GA < nch) & (t >= SLAG))
                def _():
                    sc_wait((kk + GA) % NROW)

                @pl.when(t + GA < nch)
                def _():
                    idx_wait((kk + GA) % NIDX)
                    g_start((kk + GA) % NIDX, (kk + GA) % NROW)
            return carry

        lax.fori_loop(0, nch // NIDX, body, 0)

        @pl.when(nch > 0)
        def _():
            for b in range(NROW):
                sc_wait(b)

        plsc.subcore_barrier()

        for t in range(ROWS_PER_TILE // CHUNK):
            r0 = s * ROWS_PER_TILE + t * CHUNK
            pltpu.async_copy(acc.at[pl.ds(r0, CHUNK)], out_hbm.at[c].at[pl.ds(r0, CHUNK)], sg[0])
        for t in range(ROWS_PER_TILE // CHUNK):
            r0 = s * ROWS_PER_TILE + t * CHUNK
            pltpu.make_async_copy(acc.at[pl.ds(r0, CHUNK)], out_hbm.at[c].at[pl.ds(r0, CHUNK)], sg[0]).wait()

    return k(hp, src_f, dst_f)


_BLK = 1000


def _dinv_block(deg2_ref):
    deg = deg2_ref[0, :, :1] + deg2_ref[1, :, :1] + 1.0
    return lax.rsqrt(jnp.maximum(deg, 1.0))


def _tc_first_body(deg2_ref, x_ref, w1_ref, out_ref):
    dinv = _dinv_block(deg2_ref)
    h = jnp.dot(x_ref[...], w1_ref[...], preferred_element_type=jnp.float32)
    out_ref[...] = h * dinv


def _tc_mid_body(deg2_ref, agg_ref, hp_ref, b_ref, w_ref, out_ref):
    dinv = _dinv_block(deg2_ref)
    z = dinv * (agg_ref[0] + agg_ref[1] + hp_ref[...]) + b_ref[...]
    z = jnp.where(z > 0.0, z, jnp.exp(z) - 1.0)
    out_ref[...] = jnp.dot(z, w_ref[...], preferred_element_type=jnp.float32) * dinv


def _tc_final_body(deg2_ref, agg_ref, hp_ref, b_ref, w_ref, b3_ref, out_ref):
    dinv = _dinv_block(deg2_ref)
    z = dinv * (agg_ref[0] + agg_ref[1] + hp_ref[...]) + b_ref[...]
    z = jnp.where(z > 0.0, z, jnp.exp(z) - 1.0)
    logits = jnp.dot(z, w_ref[...], preferred_element_type=jnp.float32) + b3_ref[...]
    m = jnp.max(logits, axis=1, keepdims=True)
    e = jnp.exp(logits - m)
    lse = jnp.log(jnp.sum(e, axis=1, keepdims=True)) + m
    out_ref[...] = logits - lse


def _deg2_spec():
    return pl.BlockSpec((NC, _BLK, DD), lambda i: (0, i, 0))


def _agg_spec():
    return pl.BlockSpec((NC, _BLK, DD), lambda i: (0, i, 0))


def _row_spec(d):
    return pl.BlockSpec((_BLK, d), lambda i: (i, 0))


def _full_spec(r, c):
    return pl.BlockSpec((r, c), lambda i: (0, 0))


def _tc_first(deg2, x, W1):
    return pl.pallas_call(
        _tc_first_body,
        grid=(NN // _BLK,),
        in_specs=[_deg2_spec(), _row_spec(DD), _full_spec(DD, HH)],
        out_specs=_row_spec(HH),
        out_shape=jax.ShapeDtypeStruct((NN, HH), jnp.float32),
    )(deg2, x, W1)


def _tc_mid(deg2, agg, hp, b1, W2):
    return pl.pallas_call(
        _tc_mid_body,
        grid=(NN // _BLK,),
        in_specs=[_deg2_spec(), _agg_spec(), _row_spec(HH),
                  _full_spec(1, HH), _full_spec(HH, HH)],
        out_specs=_row_spec(HH),
        out_shape=jax.ShapeDtypeStruct((NN, HH), jnp.float32),
    )(deg2, agg, hp, b1, W2)


def _tc_final(deg2, agg, hp, b2, W3, b3):
    return pl.pallas_call(
        _tc_final_body,
        grid=(NN // _BLK,),
        in_specs=[_deg2_spec(), _agg_spec(), _row_spec(HH),
                  _full_spec(1, HH), _full_spec(HH, CC), _full_spec(1, CC)],
        out_specs=_row_spec(CC),
        out_shape=jax.ShapeDtypeStruct((NN, CC), jnp.float32),
    )(deg2, agg, hp, b2, W3, b3)


def kernel(x, edge_index, W1, b1, W2, b2, W3, b3):
    E = edge_index.shape[1]
    src = edge_index[0].astype(jnp.int32)
    dst = edge_index[1].astype(jnp.int32)

    per = NWK * CHUNK
    cpw = (E + per - 1) // per
    cpw = ((cpw + NIDX - 1) // NIDX) * NIDX
    EP = cpw * per
    pad = EP - E
    if pad:
        src = jnp.concatenate([src, jnp.zeros((pad,), jnp.int32)])
        dst = jnp.concatenate([dst, jnp.full((pad,), NN, jnp.int32)])
    dst_r = dst.reshape(NWK, cpw * CHUNK)

    cpw_a = (CPW_A_FRAC * 2 * cpw // NIDX // CPW_DEN) * NIDX
    cpw_b = 2 * cpw - cpw_a

    deg2 = _sc_degree(dst_r, cpw)
    hp1 = _tc_first(deg2, x, W1)
    agg1 = _sc_segment_sum(hp1, src, dst, cpw_a, cpw_b)
    hp2 = _tc_mid(deg2, agg1, hp1, b1.reshape(1, HH), W2)
    agg2 = _sc_segment_sum(hp2, src, dst, cpw_a, cpw_b)
    return _tc_final(deg2, agg2, hp2, b2.reshape(1, HH), W3, b3.reshape(1, CC))


# speedup vs baseline: 1.0169x; 1.0107x over previous
import functools

import jax
import jax.numpy as jnp
from jax import lax
from jax.experimental import pallas as pl
from jax.experimental.pallas import tpu as pltpu
from jax.experimental.pallas import tpu_sc as plsc

NN = 10000
DD = 128
HH = 128
CC = 40

NC = 2
NS = 16
NWK = NC * NS
CHUNK = 64
NPAD = 10240
ROWS_PER_TILE = NPAD // NS

NROW = 4
NIDX = 8
GA = 3
SLAG = NROW - GA
DNROW = 4
DNIDX = 8
CPW_A_FRAC, CPW_DEN = 19, 20


def _mesh():
    return plsc.VectorSubcoreMesh(core_axis_name="c", subcore_axis_name="s")


def _zero_fill(buf, n_rows):
    def zero(g, carry):
        for j in range(DD // 16):
            buf[g, pl.ds(j * 16, 16)] = jnp.zeros((16,), jnp.float32)
        return carry

    lax.fori_loop(0, n_rows, zero, 0)


def _sc_degree(dst_r, cpw):
    @functools.partial(
        pl.kernel,
        out_type=jax.ShapeDtypeStruct((NC, NPAD, DD), jnp.float32),
        mesh=_mesh(),
        scratch_types=(
            [pltpu.VMEM((CHUNK, DD), jnp.float32)]
            + [pltpu.VMEM((CHUNK,), jnp.int32)] * DNIDX
            + [pltpu.SemaphoreType.DMA] * DNIDX
            + [pltpu.SemaphoreType.DMA] * DNROW
            + [pltpu.VMEM_SHARED((NPAD, DD), jnp.float32)]
        ),
    )
    def k(dst_hbm, out_hbm, *rest):
        ones_v = rest[0]
        idxd = rest[1:1 + DNIDX]
        si = rest[1 + DNIDX:1 + 2 * DNIDX]
        ss = rest[1 + 2 * DNIDX:1 + 2 * DNIDX + DNROW]
        acc = rest[1 + 2 * DNIDX + DNROW]

        c = lax.axis_index("c")
        s = lax.axis_index("s")
        wid = s * NC + c

        _zero_fill(ones_v, CHUNK)
        for q in range(DNIDX):
            idxd[q][pl.ds(0, 16)] = jnp.zeros((16,), jnp.int32)
        for t in range(ROWS_PER_TILE // CHUNK):
            pltpu.sync_copy(ones_v, acc.at[pl.ds(s * ROWS_PER_TILE + t * CHUNK, CHUNK)])

        def fill(g, carry):
            for j in range(DD // 16):
                ones_v[g, pl.ds(j * 16, 16)] = jnp.full((16,), 1.0, jnp.float32)
            return carry

        lax.fori_loop(0, CHUNK, fill, 0)
        plsc.subcore_barrier()

        def idx_start(t, q):
            pltpu.async_copy(dst_hbm.at[wid].at[pl.ds(t * CHUNK, CHUNK)], idxd[q], si[q])

        def idx_wait(q):
            pltpu.make_async_copy(dst_hbm.at[wid].at[pl.ds(0, CHUNK)], idxd[q], si[q]).wait()

        def sc_start(q, b):
            pltpu.async_copy(ones_v, acc.at[idxd[q]], ss[b], add=True)

        def sc_wait(b):
            pltpu.make_async_copy(ones_v, acc.at[idxd[0]], ss[b]).wait()

        for q in range(DNROW):
            idx_start(q, q)

        def body(i, carry):
            for kk in range(DNIDX):
                t = i * DNIDX + kk
                b = kk % DNROW

                @pl.when(t >= DNROW)
                def _():
                    sc_wait(b)

                @pl.when(t + DNROW < cpw)
                def _():
                    idx_start(t + DNROW, (kk + DNROW) % DNIDX)

                idx_wait(kk)
                sc_start(kk, b)
            return carry

        lax.fori_loop(0, cpw // DNIDX, body, 0)
        for b in range(DNROW):
            sc_wait(b)
        plsc.subcore_barrier()

        for t in range(ROWS_PER_TILE // CHUNK):
            r0 = s * ROWS_PER_TILE + t * CHUNK
            pltpu.async_copy(acc.at[pl.ds(r0, CHUNK)], out_hbm.at[c].at[pl.ds(r0, CHUNK)], si[0])
        for t in range(ROWS_PER_TILE // CHUNK):
            r0 = s * ROWS_PER_TILE + t * CHUNK
            pltpu.make_async_copy(acc.at[pl.ds(r0, CHUNK)], out_hbm.at[c].at[pl.ds(r0, CHUNK)], si[0]).wait()

    return k(dst_r)


def _sc_segment_sum(hp, src_f, dst_f, cpw_a, cpw_b):
    @functools.partial(
        pl.kernel,
        out_type=jax.ShapeDtypeStruct((NC, NPAD, DD), jnp.float32),
        mesh=_mesh(),
        scratch_types=(
            [pltpu.VMEM((CHUNK,), jnp.int32)] * NIDX
            + [pltpu.VMEM((CHUNK,), jnp.int32)] * NIDX
            + [pltpu.VMEM((CHUNK, DD), jnp.float32)] * NROW
            + [pltpu.SemaphoreType.DMA] * NIDX
            + [pltpu.SemaphoreType.DMA] * NROW
            + [pltpu.SemaphoreType.DMA] * NROW
            + [pltpu.VMEM_SHARED((NPAD, DD), jnp.float32)]
        ),
    )
    def k(hp_hbm, src_hbm, dst_hbm, out_hbm, *rest):
        idxs = rest[0:NIDX]
        idxd = rest[NIDX:2 * NIDX]
        rows = rest[2 * NIDX:2 * NIDX + NROW]
        si = rest[2 * NIDX + NROW:3 * NIDX + NROW]
        sg = rest[3 * NIDX + NROW:3 * NIDX + 2 * NROW]
        ss = rest[3 * NIDX + 2 * NROW:3 * NIDX + 3 * NROW]
        acc = rest[3 * NIDX + 3 * NROW]

        c = lax.axis_index("c")
        s = lax.axis_index("s")
        nch = jnp.where(c == 0, cpw_a, cpw_b)
        base_ch = jnp.where(c == 0, s * cpw_a, NS * cpw_a + s * cpw_b)

        _zero_fill(rows[0], CHUNK)
        for b in range(1, NROW):
            rows[b][0, pl.ds(0, 16)] = jnp.zeros((16,), jnp.float32)
        for q in range(NIDX):
            idxs[q][pl.ds(0, 16)] = jnp.zeros((16,), jnp.int32)
            idxd[q][pl.ds(0, 16)] = jnp.zeros((16,), jnp.int32)
        for t in range(ROWS_PER_TILE // CHUNK):
            pltpu.sync_copy(rows[0], acc.at[pl.ds(s * ROWS_PER_TILE + t * CHUNK, CHUNK)])
        plsc.subcore_barrier()

        def idx_start(t, q):
            e0 = (base_ch + t) * CHUNK
            pltpu.async_copy(src_hbm.at[pl.ds(e0, CHUNK)], idxs[q], si[q])
            pltpu.async_copy(dst_hbm.at[pl.ds(e0, CHUNK)], idxd[q], si[q])

        def idx_wait(q):
            pltpu.make_async_copy(src_hbm.at[pl.ds(0, CHUNK)], idxs[q], si[q]).wait()
            pltpu.make_async_copy(dst_hbm.at[pl.ds(0, CHUNK)], idxd[q], si[q]).wait()

        def g_start(q, b):
            pltpu.async_copy(hp_hbm.at[idxs[q]], rows[b], sg[b])

        def g_wait(q, b):
            pltpu.make_async_copy(hp_hbm.at[idxs[q]], rows[b], sg[b]).wait()

        def sc_start(q, b):
            pltpu.async_copy(rows[b], acc.at[idxd[q]], ss[b], add=True)

        def sc_wait(b):
            pltpu.make_async_copy(rows[b], acc.at[idxd[0]], ss[b]).wait()

        @pl.when(nch > 0)
        def _():
            for q in range(NROW):
                idx_start(q, q)
            for t in range(GA):
                idx_wait(t)
                g_start(t, t)

        def body(i, carry):
            for kk in range(NIDX):
                t = i * NIDX + kk
                b = kk % NROW

                g_wait(kk, b)
                sc_start(kk, b)

                @pl.when(t + NROW < nch)
                def _():
                    idx_start(t + NROW, (kk + NROW) % NIDX)

                @pl.when((t + GA < nch) & (t >= SLAG))
                def _():
                    sc_wait((kk + GA) % NROW)

                @pl.when(t + GA < nch)
                def _():
                    idx_wait((kk + GA) % NIDX)
                    g_start((kk + GA) % NIDX, (kk + GA) % NROW)
            return carry

        lax.fori_loop(0, nch // NIDX, body, 0)

        @pl.when(nch > 0)
        def _():
            for b in range(NROW):
                sc_wait(b)

        plsc.subcore_barrier()

        for t in range(ROWS_PER_TILE // CHUNK):
            r0 = s * ROWS_PER_TILE + t * CHUNK
            pltpu.async_copy(acc.at[pl.ds(r0, CHUNK)], out_hbm.at[c].at[pl.ds(r0, CHUNK)], sg[0])
        for t in range(ROWS_PER_TILE // CHUNK):
            r0 = s * ROWS_PER_TILE + t * CHUNK
            pltpu.make_async_copy(acc.at[pl.ds(r0, CHUNK)], out_hbm.at[c].at[pl.ds(r0, CHUNK)], sg[0]).wait()

    return k(hp, src_f, dst_f)


_BLK = 1000


def _dinv_block(deg2_ref):
    deg = deg2_ref[0, :, :1] + deg2_ref[1, :, :1] + 1.0
    return lax.rsqrt(jnp.maximum(deg, 1.0))


def _tc_first_body(deg2_ref, x_ref, w1_ref, out_ref):
    dinv = _dinv_block(deg2_ref)
    h = jnp.dot(x_ref[...], w1_ref[...], preferred_element_type=jnp.float32)
    out_ref[...] = h * dinv


def _tc_mid_body(deg2_ref, agg_ref, hp_ref, b_ref, w_ref, out_ref):
    dinv = _dinv_block(deg2_ref)
    z = dinv * (agg_ref[0] + agg_ref[1] + hp_ref[...]) + b_ref[...]
    z = jnp.where(z > 0.0, z, jnp.exp(z) - 1.0)
    out_ref[...] = jnp.dot(z, w_ref[...], preferred_element_type=jnp.float32) * dinv


def _tc_final_body(deg2_ref, agg_ref, hp_ref, b_ref, w_ref, b3_ref, out_ref):
    dinv = _dinv_block(deg2_ref)
    z = dinv * (agg_ref[0] + agg_ref[1] + hp_ref[...]) + b_ref[...]
    z = jnp.where(z > 0.0, z, jnp.exp(z) - 1.0)
    logits = jnp.dot(z, w_ref[...], preferred_element_type=jnp.float32) + b3_ref[...]
    m = jnp.max(logits, axis=1, keepdims=True)
    e = jnp.exp(logits - m)
    lse = jnp.log(jnp.sum(e, axis=1, keepdims=True)) + m
    out_ref[...] = logits - lse


def _deg2_spec():
    return pl.BlockSpec((NC, _BLK, DD), lambda i: (0, i, 0))


def _agg_spec():
    return pl.BlockSpec((NC, _BLK, DD), lambda i: (0, i, 0))


def _row_spec(d):
    return pl.BlockSpec((_BLK, d), lambda i: (i, 0))


def _full_spec(r, c):
    return pl.BlockSpec((r, c), lambda i: (0, 0))


def _tc_first(deg2, x, W1):
    return pl.pallas_call(
        _tc_first_body,
        grid=(NN // _BLK,),
        in_specs=[_deg2_spec(), _row_spec(DD), _full_spec(DD, HH)],
        out_specs=_row_spec(HH),
        out_shape=jax.ShapeDtypeStruct((NN, HH), jnp.float32),
    )(deg2, x, W1)


def _tc_mid(deg2, agg, hp, b1, W2):
    return pl.pallas_call(
        _tc_mid_body,
        grid=(NN // _BLK,),
        in_specs=[_deg2_spec(), _agg_spec(), _row_spec(HH),
                  _full_spec(1, HH), _full_spec(HH, HH)],
        out_specs=_row_spec(HH),
        out_shape=jax.ShapeDtypeStruct((NN, HH), jnp.float32),
    )(deg2, agg, hp, b1, W2)


def _tc_final(deg2, agg, hp, b2, W3, b3):
    return pl.pallas_call(
        _tc_final_body,
        grid=(NN // _BLK,),
        in_specs=[_deg2_spec(), _agg_spec(), _row_spec(HH),
                  _full_spec(1, HH), _full_spec(HH, CC), _full_spec(1, CC)],
        out_specs=_row_spec(CC),
        out_shape=jax.ShapeDtypeStruct((NN, CC), jnp.float32),
    )(deg2, agg, hp, b2, W3, b3)


def kernel(x, edge_index, W1, b1, W2, b2, W3, b3):
    E = edge_index.shape[1]
    src = edge_index[0].astype(jnp.int32)
    dst = edge_index[1].astype(jnp.int32)

    per = NWK * CHUNK
    cpw = (E + per - 1) // per
    cpw = ((cpw + NIDX - 1) // NIDX) * NIDX
    EP = cpw * per
    pad = EP - E
    if pad:
        src = jnp.concatenate([src, jnp.zeros((pad,), jnp.int32)])
        dst = jnp.concatenate([dst, jnp.full((pad,), NN, jnp.int32)])
    dst_r = dst.reshape(NWK, cpw * CHUNK)

    cpw_a = (CPW_A_FRAC * 2 * cpw // NIDX // CPW_DEN) * NIDX
    cpw_b = 2 * cpw - cpw_a

    deg2 = _sc_degree(dst_r, cpw)
    hp1 = _tc_first(deg2, x, W1)
    agg1 = _sc_segment_sum(hp1, src, dst, cpw_a, cpw_b)
    hp2 = _tc_mid(deg2, agg1, hp1, b1.reshape(1, HH), W2)
    agg2 = _sc_segment_sum(hp2, src, dst, cpw_a, cpw_b)
    return _tc_final(deg2, agg2, hp2, b2.reshape(1, HH), W3, b3.reshape(1, CC))
